# Initial kernel scaffold; baseline (speedup 1.0000x reference)
import jax
import jax.numpy as jnp
from jax.experimental import pallas as pl


def kernel(x, edge_index, W1, att_src1, att_dst1, b1, W2, att_src2, att_dst2, b2, W3, att_src3, att_dst3, b3, lin1_W, lin1_b, lin2_W, lin2_b):
    raise NotImplementedError("write your pallas kernel here")


# speedup vs baseline: 24.5343x; 24.5343x over previous
import dataclasses
import functools

import jax
import jax.numpy as jnp
from jax import lax
from jax.experimental import pallas as pl
from jax.experimental.pallas import tpu as pltpu
from jax.experimental.pallas import tpu_sc as plsc

N = 10000
E = 320000
D = 128
NC = 2
NS = 16
L = 16
NW = NC * NS
EPW = E // NW
B = 96
NBLK = 105
PAD_PER_W = NBLK * B - EPW
NPAD = 10112
TRASH = NPAD - N
RB = 1000
NRB = N // RB
SHIFT = 30.0

_f32 = jnp.float32


def _attention_stats(h, asv, adv, i, asrc_ref, adst_ref, m_ref, sa_ref, sd_ref):
    a_s = jnp.sum(h * asv, axis=1)
    a_d = jnp.sum(h * adv, axis=1)
    asrc_ref[...] = a_s.reshape(1, 1, RB)
    adst_ref[...] = a_d.reshape(1, 1, RB)
    bs = jnp.max(a_s)
    bd = jnp.max(a_d)

    @pl.when(i == 0)
    def _():
        sa_ref[0] = bs
        sd_ref[0] = bd

    @pl.when(i > 0)
    def _():
        sa_ref[0] = jnp.maximum(sa_ref[0], bs)
        sd_ref[0] = jnp.maximum(sd_ref[0], bd)

    @pl.when(i == NRB - 1)
    def _():
        m_ref[...] = jnp.full((1, D), sa_ref[0] + sd_ref[0] - SHIFT, _f32)


def _prep1_body(x_ref, w_ref, asv_ref, adv_ref,
                h_ref, asrc_ref, adst_ref, m_ref, sa_ref, sd_ref):
    i = pl.program_id(0)
    h = jnp.dot(x_ref[...], w_ref[...], preferred_element_type=_f32)
    h_ref[...] = h
    _attention_stats(h, asv_ref[...], adv_ref[...], i,
                     asrc_ref, adst_ref, m_ref, sa_ref, sd_ref)


def _prepmid_body(acc_ref, den_ref, b_ref, w_ref, asv_ref, adv_ref,
                  h_ref, asrc_ref, adst_ref, m_ref, sa_ref, sd_ref):
    i = pl.program_id(0)
    a = acc_ref[0] + acc_ref[1]
    dnm = den_ref[0, 0, 0] + den_ref[1, 0, 0] + 1e-16
    z = jnp.maximum(a / dnm[:, None] + b_ref[...], 0.0)
    h = jnp.dot(z, w_ref[...], preferred_element_type=_f32)
    h_ref[...] = h
    _attention_stats(h, asv_ref[...], adv_ref[...], i,
                     asrc_ref, adst_ref, m_ref, sa_ref, sd_ref)


def _final_body(acc_ref, den_ref, b_ref, w1_ref, b1_ref, w2_ref, b2_ref, out_ref):
    a = acc_ref[0] + acc_ref[1]
    dnm = den_ref[0, 0, 0] + den_ref[1, 0, 0] + 1e-16
    z = jnp.maximum(a / dnm[:, None] + b_ref[...], 0.0)
    t = jnp.maximum(
        jnp.dot(z, w1_ref[...], preferred_element_type=_f32) + b1_ref[...], 0.0)
    out_ref[...] = (
        jnp.dot(t, w2_ref[...], preferred_element_type=_f32) + b2_ref[...])


_row_spec = pl.BlockSpec((RB, D), lambda i: (i, 0))
_mat_spec = pl.BlockSpec((D, D), lambda i: (0, 0))
_vec_spec = pl.BlockSpec((1, D), lambda i: (0, 0))
_a_spec = pl.BlockSpec((1, 1, RB), lambda i: (i, 0, 0))
_acc_spec = pl.BlockSpec((NC, RB, D), lambda i: (0, i, 0))
_den_spec = pl.BlockSpec((NC, 1, 1, RB), lambda i: (0, i, 0, 0))

_prep_outs = dict(
    out_specs=[_row_spec, _a_spec, _a_spec, _vec_spec],
    out_shape=[
        jax.ShapeDtypeStruct((N, D), _f32),
        jax.ShapeDtypeStruct((NRB, 1, RB), _f32),
        jax.ShapeDtypeStruct((NRB, 1, RB), _f32),
        jax.ShapeDtypeStruct((1, D), _f32),
    ],
    scratch_shapes=[pltpu.SMEM((1,), _f32), pltpu.SMEM((1,), _f32)],
)

_prep1 = pl.pallas_call(
    _prep1_body,
    grid=(NRB,),
    in_specs=[_row_spec, _mat_spec, _vec_spec, _vec_spec],
    **_prep_outs,
)

_prepmid = pl.pallas_call(
    _prepmid_body,
    grid=(NRB,),
    in_specs=[_acc_spec, _den_spec, _vec_spec, _mat_spec, _vec_spec, _vec_spec],
    **_prep_outs,
)

_final = pl.pallas_call(
    _final_body,
    grid=(NRB,),
    in_specs=[_acc_spec, _den_spec, _vec_spec, _mat_spec, _vec_spec,
              _mat_spec, _vec_spec],
    out_specs=_row_spec,
    out_shape=jax.ShapeDtypeStruct((N, D), _f32),
)


_sc_mesh = plsc.VectorSubcoreMesh(core_axis_name="c", subcore_axis_name="s")

_sc_cp = pltpu.CompilerParams()
if "needs_layout_passes" in pltpu.CompilerParams.__dataclass_fields__:
    _sc_cp = dataclasses.replace(_sc_cp, needs_layout_passes=False)


@functools.partial(
    pl.kernel,
    compiler_params=_sc_cp,
    out_type=[
        jax.ShapeDtypeStruct((NC, N, D), _f32),
        jax.ShapeDtypeStruct((NC * N,), _f32),
    ],
    mesh=_sc_mesh,
    scratch_types=[
        pltpu.VMEM((B,), _f32),
        pltpu.VMEM((B,), _f32),
        pltpu.VMEM((B,), jnp.int32),
        pltpu.VMEM((B,), jnp.int32),
        pltpu.VMEM((B,), _f32),
        pltpu.VMEM((B, D), _f32),
        pltpu.VMEM((640,), _f32),
        pltpu.VMEM((1, D), _f32),
        pltpu.VMEM_SHARED((NPAD, D), _f32),
        pltpu.VMEM_SHARED((NPAD,), _f32),
        pltpu.SemaphoreType.DMA,
    ],
)
def _edge_pass(h_hbm, asrc_hbm, adst_hbm, srcp_hbm, dstp_hbm, m_hbm, zacc_hbm,
               accs_out, dens_out,
               asrc_v, adst_v, src_v, dst_v, ex_v, rows_v, den_stage, m_s,
               acc_sh, den_sh, sem):
    cid = lax.axis_index("c")
    sid = lax.axis_index("s")
    wid = sid * NC + cid

    pltpu.sync_copy(m_hbm, m_s)
    m = m_s[0, pl.ds(0, L)][0]

    zr = NPAD // NS
    pltpu.sync_copy(zacc_hbm.at[pl.ds(sid * zr, zr)],
                    acc_sh.at[pl.ds(sid * zr, zr)])
    @pl.loop(0, 40)
    def _(j):
        den_stage[pl.ds(j * L, L)] = jnp.zeros((L,), _f32)

    pltpu.sync_copy(den_stage.at[pl.ds(0, zr)], den_sh.at[pl.ds(sid * zr, zr)])

    plsc.subcore_barrier()

    @pl.loop(0, NBLK)
    def _(blk):
        eb = wid * (NBLK * B) + blk * B
        pltpu.sync_copy(srcp_hbm.at[pl.ds(eb, B)], src_v)
        pltpu.sync_copy(dstp_hbm.at[pl.ds(eb, B)], dst_v)
        gat = pltpu.async_copy(h_hbm.at[src_v], rows_v, sem)
        pltpu.sync_copy(asrc_hbm.at[src_v], asrc_v)
        pltpu.sync_copy(adst_hbm.at[dst_v], adst_v)

        @pl.loop(0, B // L)
        def _(g):
            sl = pl.ds(g * L, L)
            e = asrc_v[sl] + adst_v[sl]
            e = jnp.where(e >= 0.0, e, 0.2 * e)
            ex_v[sl] = jnp.exp(e - m)

        gat.wait()

        @pl.loop(0, B // L)
        def _(g):
            exg = ex_v[pl.ds(g * L, L)]
            for k in range(L):
                j = g * L + k
                exv = jnp.full((L,), exg[k], _f32)
                for c in range(D // L):
                    sl = pl.ds(c * L, L)
                    rows_v[j, sl] = rows_v[j, sl] * exv

        pltpu.sync_copy(rows_v, acc_sh.at[dst_v], add=True)
        pltpu.sync_copy(ex_v, den_sh.at[dst_v], add=True)

    plsc.subcore_barrier()

    orows = (N // NS) // 8 * 8
    pltpu.sync_copy(acc_sh.at[pl.ds(sid * orows, orows)],
                    accs_out.at[cid, pl.ds(sid * orows, orows)])

    @pl.when(sid == 0)
    def _():
        pltpu.sync_copy(acc_sh.at[pl.ds(NS * orows, N - NS * orows)],
                        accs_out.at[cid, pl.ds(NS * orows, N - NS * orows)])

    pltpu.sync_copy(den_sh.at[pl.ds(sid * orows, orows)],
                    den_stage.at[pl.ds(0, orows)])
    pltpu.sync_copy(den_stage.at[pl.ds(0, orows)],
                    dens_out.at[pl.ds(cid * N + sid * orows, orows)])

    @pl.when(sid == 0)
    def _():
        rem = N - NS * orows
        pltpu.sync_copy(den_sh.at[pl.ds(NS * orows, rem)],
                        den_stage.at[pl.ds(orows, rem)])
        pltpu.sync_copy(den_stage.at[pl.ds(orows, rem)],
                        dens_out.at[pl.ds(cid * N + NS * orows, rem)])


def _edge(h, asrc, adst, srcp, dstp, mv, zacc):
    asrc_e = jnp.pad(asrc.reshape(N), (0, TRASH))
    adst_e = jnp.pad(adst.reshape(N), (0, TRASH))
    accs, dens_flat = _edge_pass(h, asrc_e, adst_e, srcp, dstp, mv, zacc)
    dens = dens_flat.reshape(NC, NRB, 1, RB)
    return accs, dens


def kernel(x, edge_index, W1, att_src1, att_dst1, b1, W2, att_src2, att_dst2,
           b2, W3, att_src3, att_dst3, b3, lin1_W, lin1_b, lin2_W, lin2_b):
    src = edge_index[0].astype(jnp.int32).reshape(NW, EPW)
    dst = edge_index[1].astype(jnp.int32).reshape(NW, EPW)
    pads = jnp.arange(PAD_PER_W, dtype=jnp.int32) % L
    padd = N + jnp.arange(PAD_PER_W, dtype=jnp.int32) % TRASH
    srcp = jnp.concatenate(
        [src, jnp.broadcast_to(pads[None], (NW, PAD_PER_W))],
        axis=1).reshape(NW * NBLK * B)
    dstp = jnp.concatenate(
        [dst, jnp.broadcast_to(padd[None], (NW, PAD_PER_W))],
        axis=1).reshape(NW * NBLK * B)
    zacc = jnp.zeros((NPAD, D), _f32)

    h, asrc, adst, mv = _prep1(x, W1, att_src1.reshape(1, D),
                               att_dst1.reshape(1, D))
    accs, dens = _edge(h, asrc, adst, srcp, dstp, mv, zacc)
    h, asrc, adst, mv = _prepmid(accs, dens, b1.reshape(1, D), W2,
                                 att_src2.reshape(1, D), att_dst2.reshape(1, D))
    accs, dens = _edge(h, asrc, adst, srcp, dstp, mv, zacc)
    h, asrc, adst, mv = _prepmid(accs, dens, b2.reshape(1, D), W3,
                                 att_src3.reshape(1, D), att_dst3.reshape(1, D))
    accs, dens = _edge(h, asrc, adst, srcp, dstp, mv, zacc)
    out = _final(accs, dens, b3.reshape(1, D), lin1_W, lin1_b.reshape(1, D),
                 lin2_W, lin2_b.reshape(1, D))
    return out


# speedup vs baseline: 36.1004x; 1.4714x over previous
import dataclasses
import functools

import jax
import jax.numpy as jnp
from jax import lax
from jax.experimental import pallas as pl
from jax.experimental.pallas import tpu as pltpu
from jax.experimental.pallas import tpu_sc as plsc

N = 10000
E = 320000
D = 128
NC = 2
NS = 16
L = 16
NW = NC * NS
EPW = E // NW
B = 96
NBLK = 106
PAD_PER_W = NBLK * B - EPW
NPAD = 10112
TRASH = NPAD - N
RB = 1000
NRB = N // RB
SHIFT = 30.0

_f32 = jnp.float32


def _attention_stats(h, asv, adv, i, asrc_ref, adst_ref, m_ref, sa_ref, sd_ref):
    a_s = jnp.sum(h * asv, axis=1)
    a_d = jnp.sum(h * adv, axis=1)
    asrc_ref[...] = a_s.reshape(1, 1, RB)
    adst_ref[...] = a_d.reshape(1, 1, RB)
    bs = jnp.max(a_s)
    bd = jnp.max(a_d)

    @pl.when(i == 0)
    def _():
        sa_ref[0] = bs
        sd_ref[0] = bd

    @pl.when(i > 0)
    def _():
        sa_ref[0] = jnp.maximum(sa_ref[0], bs)
        sd_ref[0] = jnp.maximum(sd_ref[0], bd)

    @pl.when(i == NRB - 1)
    def _():
        m_ref[...] = jnp.full((1, D), sa_ref[0] + sd_ref[0] - SHIFT, _f32)


def _prep1_body(x_ref, w_ref, asv_ref, adv_ref,
                h_ref, asrc_ref, adst_ref, m_ref, sa_ref, sd_ref):
    i = pl.program_id(0)
    h = jnp.dot(x_ref[...], w_ref[...], preferred_element_type=_f32)
    h_ref[...] = h
    _attention_stats(h, asv_ref[...], adv_ref[...], i,
                     asrc_ref, adst_ref, m_ref, sa_ref, sd_ref)


def _prepmid_body(acc_ref, den_ref, b_ref, w_ref, asv_ref, adv_ref,
                  h_ref, asrc_ref, adst_ref, m_ref, sa_ref, sd_ref):
    i = pl.program_id(0)
    a = acc_ref[0] + acc_ref[1]
    dnm = den_ref[0, 0, 0] + den_ref[1, 0, 0] + 1e-16
    z = jnp.maximum(a / dnm[:, None] + b_ref[...], 0.0)
    h = jnp.dot(z, w_ref[...], preferred_element_type=_f32)
    h_ref[...] = h
    _attention_stats(h, asv_ref[...], adv_ref[...], i,
                     asrc_ref, adst_ref, m_ref, sa_ref, sd_ref)


def _final_body(acc_ref, den_ref, b_ref, w1_ref, b1_ref, w2_ref, b2_ref, out_ref):
    a = acc_ref[0] + acc_ref[1]
    dnm = den_ref[0, 0, 0] + den_ref[1, 0, 0] + 1e-16
    z = jnp.maximum(a / dnm[:, None] + b_ref[...], 0.0)
    t = jnp.maximum(
        jnp.dot(z, w1_ref[...], preferred_element_type=_f32) + b1_ref[...], 0.0)
    out_ref[...] = (
        jnp.dot(t, w2_ref[...], preferred_element_type=_f32) + b2_ref[...])


_row_spec = pl.BlockSpec((RB, D), lambda i: (i, 0))
_mat_spec = pl.BlockSpec((D, D), lambda i: (0, 0))
_vec_spec = pl.BlockSpec((1, D), lambda i: (0, 0))
_a_spec = pl.BlockSpec((1, 1, RB), lambda i: (i, 0, 0))
_acc_spec = pl.BlockSpec((NC, RB, D), lambda i: (0, i, 0))
_den_spec = pl.BlockSpec((NC, 1, 1, RB), lambda i: (0, i, 0, 0))

_prep_outs = dict(
    out_specs=[_row_spec, _a_spec, _a_spec, _vec_spec],
    out_shape=[
        jax.ShapeDtypeStruct((N, D), _f32),
        jax.ShapeDtypeStruct((NRB, 1, RB), _f32),
        jax.ShapeDtypeStruct((NRB, 1, RB), _f32),
        jax.ShapeDtypeStruct((1, D), _f32),
    ],
    scratch_shapes=[pltpu.SMEM((1,), _f32), pltpu.SMEM((1,), _f32)],
)

_prep1 = pl.pallas_call(
    _prep1_body,
    grid=(NRB,),
    in_specs=[_row_spec, _mat_spec, _vec_spec, _vec_spec],
    **_prep_outs,
)

_prepmid = pl.pallas_call(
    _prepmid_body,
    grid=(NRB,),
    in_specs=[_acc_spec, _den_spec, _vec_spec, _mat_spec, _vec_spec, _vec_spec],
    **_prep_outs,
)

_final = pl.pallas_call(
    _final_body,
    grid=(NRB,),
    in_specs=[_acc_spec, _den_spec, _vec_spec, _mat_spec, _vec_spec,
              _mat_spec, _vec_spec],
    out_specs=_row_spec,
    out_shape=jax.ShapeDtypeStruct((N, D), _f32),
)


_sc_mesh = plsc.VectorSubcoreMesh(core_axis_name="c", subcore_axis_name="s")

_sc_cp = pltpu.CompilerParams()
if "needs_layout_passes" in pltpu.CompilerParams.__dataclass_fields__:
    _sc_cp = dataclasses.replace(_sc_cp, needs_layout_passes=False)


@functools.partial(
    pl.kernel,
    compiler_params=_sc_cp,
    out_type=[
        jax.ShapeDtypeStruct((NC, N, D), _f32),
        jax.ShapeDtypeStruct((NC * N,), _f32),
    ],
    mesh=_sc_mesh,
    scratch_types=[
        pltpu.VMEM((2, B), _f32),
        pltpu.VMEM((2, B), _f32),
        pltpu.VMEM((2, B), jnp.int32),
        pltpu.VMEM((2, B), jnp.int32),
        pltpu.VMEM((2, B), _f32),
        pltpu.VMEM((2, B, D), _f32),
        pltpu.VMEM((640,), _f32),
        pltpu.VMEM((1, D), _f32),
        pltpu.VMEM_SHARED((NPAD, D), _f32),
        pltpu.VMEM_SHARED((NPAD,), _f32),
        pltpu.SemaphoreType.DMA,
        pltpu.SemaphoreType.DMA,
        pltpu.SemaphoreType.DMA,
        pltpu.SemaphoreType.DMA,
    ],
)
def _edge_pass(h_hbm, asrc_hbm, adst_hbm, srcp_hbm, dstp_hbm, m_hbm, zacc_hbm,
               accs_out, dens_out,
               asrc_v, adst_v, src_v, dst_v, ex_v, rows_v, den_stage, m_s,
               acc_sh, den_sh, gsem0, gsem1, asem0, asem1):
    cid = lax.axis_index("c")
    sid = lax.axis_index("s")
    wid = sid * NC + cid

    pltpu.sync_copy(m_hbm, m_s)
    m = m_s[0, pl.ds(0, L)][0]

    zr = NPAD // NS
    pltpu.sync_copy(zacc_hbm.at[pl.ds(sid * zr, zr)],
                    acc_sh.at[pl.ds(sid * zr, zr)])
    @pl.loop(0, 40)
    def _(j):
        den_stage[pl.ds(j * L, L)] = jnp.zeros((L,), _f32)

    pltpu.sync_copy(den_stage.at[pl.ds(0, zr)], den_sh.at[pl.ds(sid * zr, zr)])

    plsc.subcore_barrier()

    gsem = (gsem0, gsem1)
    asem = (asem0, asem1)
    ebase = wid * (NBLK * B)

    def _start_loads(b, q):
        eb = ebase + b * B
        pltpu.sync_copy(srcp_hbm.at[pl.ds(eb, B)], src_v.at[q])
        pltpu.sync_copy(dstp_hbm.at[pl.ds(eb, B)], dst_v.at[q])
        pltpu.async_copy(h_hbm.at[src_v.at[q]], rows_v.at[q], gsem[q])
        pltpu.async_copy(asrc_hbm.at[src_v.at[q]], asrc_v.at[q], asem[q])
        pltpu.async_copy(adst_hbm.at[dst_v.at[q]], adst_v.at[q], asem[q])

    def _process(b, p):
        q = 1 - p

        @pl.when(b + 1 < NBLK)
        def _():
            _start_loads(b + 1, q)

        pltpu.make_async_copy(asrc_hbm.at[src_v.at[p]], asrc_v.at[p],
                              asem[p]).wait()
        pltpu.make_async_copy(adst_hbm.at[dst_v.at[p]], adst_v.at[p],
                              asem[p]).wait()

        @pl.loop(0, B // L)
        def _(g):
            sl = pl.ds(g * L, L)
            e = asrc_v[p, sl] + adst_v[p, sl]
            e = jnp.where(e >= 0.0, e, 0.2 * e)
            ex_v[p, sl] = jnp.exp(e - m)

        pltpu.make_async_copy(h_hbm.at[src_v.at[p]], rows_v.at[p],
                              gsem[p]).wait()

        @pl.loop(0, B // L)
        def _(g):
            exg = ex_v[p, pl.ds(g * L, L)]
            for k in range(L):
                j = g * L + k
                exv = jnp.full((L,), exg[k], _f32)
                for c in range(D // L):
                    sl = pl.ds(c * L, L)
                    rows_v[p, j, sl] = rows_v[p, j, sl] * exv

        pltpu.sync_copy(rows_v.at[p], acc_sh.at[dst_v.at[p]], add=True)
        pltpu.sync_copy(ex_v.at[p], den_sh.at[dst_v.at[p]], add=True)

    _start_loads(0, 0)

    @pl.loop(0, NBLK, step=2)
    def _(blk):
        _process(blk, 0)
        _process(blk + 1, 1)

    plsc.subcore_barrier()

    orows = (N // NS) // 8 * 8
    pltpu.sync_copy(acc_sh.at[pl.ds(sid * orows, orows)],
                    accs_out.at[cid, pl.ds(sid * orows, orows)])

    @pl.when(sid == 0)
    def _():
        pltpu.sync_copy(acc_sh.at[pl.ds(NS * orows, N - NS * orows)],
                        accs_out.at[cid, pl.ds(NS * orows, N - NS * orows)])

    pltpu.sync_copy(den_sh.at[pl.ds(sid * orows, orows)],
                    den_stage.at[pl.ds(0, orows)])
    pltpu.sync_copy(den_stage.at[pl.ds(0, orows)],
                    dens_out.at[pl.ds(cid * N + sid * orows, orows)])

    @pl.when(sid == 0)
    def _():
        rem = N - NS * orows
        pltpu.sync_copy(den_sh.at[pl.ds(NS * orows, rem)],
                        den_stage.at[pl.ds(orows, rem)])
        pltpu.sync_copy(den_stage.at[pl.ds(orows, rem)],
                        dens_out.at[pl.ds(cid * N + NS * orows, rem)])


def _edge(h, asrc, adst, srcp, dstp, mv, zacc):
    asrc_e = jnp.pad(asrc.reshape(N), (0, TRASH))
    adst_e = jnp.pad(adst.reshape(N), (0, TRASH))
    accs, dens_flat = _edge_pass(h, asrc_e, adst_e, srcp, dstp, mv, zacc)
    dens = dens_flat.reshape(NC, NRB, 1, RB)
    return accs, dens


def kernel(x, edge_index, W1, att_src1, att_dst1, b1, W2, att_src2, att_dst2,
           b2, W3, att_src3, att_dst3, b3, lin1_W, lin1_b, lin2_W, lin2_b):
    src = edge_index[0].astype(jnp.int32).reshape(NW, EPW)
    dst = edge_index[1].astype(jnp.int32).reshape(NW, EPW)
    pads = jnp.arange(PAD_PER_W, dtype=jnp.int32) % L
    padd = N + jnp.arange(PAD_PER_W, dtype=jnp.int32) % TRASH
    srcp = jnp.concatenate(
        [src, jnp.broadcast_to(pads[None], (NW, PAD_PER_W))],
        axis=1).reshape(NW * NBLK * B)
    dstp = jnp.concatenate(
        [dst, jnp.broadcast_to(padd[None], (NW, PAD_PER_W))],
        axis=1).reshape(NW * NBLK * B)
    zacc = jnp.zeros((NPAD, D), _f32)

    h, asrc, adst, mv = _prep1(x, W1, att_src1.reshape(1, D),
                               att_dst1.reshape(1, D))
    accs, dens = _edge(h, asrc, adst, srcp, dstp, mv, zacc)
    h, asrc, adst, mv = _prepmid(accs, dens, b1.reshape(1, D), W2,
                                 att_src2.reshape(1, D), att_dst2.reshape(1, D))
    accs, dens = _edge(h, asrc, adst, srcp, dstp, mv, zacc)
    h, asrc, adst, mv = _prepmid(accs, dens, b2.reshape(1, D), W3,
                                 att_src3.reshape(1, D), att_dst3.reshape(1, D))
    accs, dens = _edge(h, asrc, adst, srcp, dstp, mv, zacc)
    out = _final(accs, dens, b3.reshape(1, D), lin1_W, lin1_b.reshape(1, D),
                 lin2_W, lin2_b.reshape(1, D))
    return out


# speedup vs baseline: 37.4249x; 1.0367x over previous
import dataclasses
import functools

import jax
import jax.numpy as jnp
from jax import lax
from jax.experimental import pallas as pl
from jax.experimental.pallas import tpu as pltpu
from jax.experimental.pallas import tpu_sc as plsc

N = 10000
E = 320000
D = 128
NC = 2
NS = 16
L = 16
NW = NC * NS
EPW = E // NW
B = 128
NBLK = 80
PAD_PER_W = NBLK * B - EPW
NPAD = 10112
TRASH = NPAD - N
RB = 1000
NRB = N // RB
SHIFT = 30.0

_f32 = jnp.float32


def _attention_stats(h, asv, adv, i, asrc_ref, adst_ref, m_ref, sa_ref, sd_ref):
    a_s = jnp.sum(h * asv, axis=1)
    a_d = jnp.sum(h * adv, axis=1)
    asrc_ref[...] = a_s.reshape(1, 1, RB)
    adst_ref[...] = a_d.reshape(1, 1, RB)
    bs = jnp.max(a_s)
    bd = jnp.max(a_d)

    @pl.when(i == 0)
    def _():
        sa_ref[0] = bs
        sd_ref[0] = bd

    @pl.when(i > 0)
    def _():
        sa_ref[0] = jnp.maximum(sa_ref[0], bs)
        sd_ref[0] = jnp.maximum(sd_ref[0], bd)

    @pl.when(i == NRB - 1)
    def _():
        m_ref[...] = jnp.full((1, D), sa_ref[0] + sd_ref[0] - SHIFT, _f32)


def _prep1_body(x_ref, w_ref, asv_ref, adv_ref,
                h_ref, asrc_ref, adst_ref, m_ref, sa_ref, sd_ref):
    i = pl.program_id(0)
    h = jnp.dot(x_ref[...], w_ref[...], preferred_element_type=_f32)
    h_ref[...] = h
    _attention_stats(h, asv_ref[...], adv_ref[...], i,
                     asrc_ref, adst_ref, m_ref, sa_ref, sd_ref)


def _prepmid_body(acc_ref, den_ref, b_ref, w_ref, asv_ref, adv_ref,
                  h_ref, asrc_ref, adst_ref, m_ref, sa_ref, sd_ref):
    i = pl.program_id(0)
    a = acc_ref[0] + acc_ref[1]
    dnm = den_ref[0, 0, 0] + den_ref[1, 0, 0] + 1e-16
    z = jnp.maximum(a / dnm[:, None] + b_ref[...], 0.0)
    h = jnp.dot(z, w_ref[...], preferred_element_type=_f32)
    h_ref[...] = h
    _attention_stats(h, asv_ref[...], adv_ref[...], i,
                     asrc_ref, adst_ref, m_ref, sa_ref, sd_ref)


def _final_body(acc_ref, den_ref, b_ref, w1_ref, b1_ref, w2_ref, b2_ref, out_ref):
    a = acc_ref[0] + acc_ref[1]
    dnm = den_ref[0, 0, 0] + den_ref[1, 0, 0] + 1e-16
    z = jnp.maximum(a / dnm[:, None] + b_ref[...], 0.0)
    t = jnp.maximum(
        jnp.dot(z, w1_ref[...], preferred_element_type=_f32) + b1_ref[...], 0.0)
    out_ref[...] = (
        jnp.dot(t, w2_ref[...], preferred_element_type=_f32) + b2_ref[...])


_row_spec = pl.BlockSpec((RB, D), lambda i: (i, 0))
_mat_spec = pl.BlockSpec((D, D), lambda i: (0, 0))
_vec_spec = pl.BlockSpec((1, D), lambda i: (0, 0))
_a_spec = pl.BlockSpec((1, 1, RB), lambda i: (i, 0, 0))
_acc_spec = pl.BlockSpec((NC, RB, D), lambda i: (0, i, 0))
_den_spec = pl.BlockSpec((NC, 1, 1, RB), lambda i: (0, i, 0, 0))

_prep_outs = dict(
    out_specs=[_row_spec, _a_spec, _a_spec, _vec_spec],
    out_shape=[
        jax.ShapeDtypeStruct((N, D), _f32),
        jax.ShapeDtypeStruct((NRB, 1, RB), _f32),
        jax.ShapeDtypeStruct((NRB, 1, RB), _f32),
        jax.ShapeDtypeStruct((1, D), _f32),
    ],
    scratch_shapes=[pltpu.SMEM((1,), _f32), pltpu.SMEM((1,), _f32)],
)

_prep1 = pl.pallas_call(
    _prep1_body,
    grid=(NRB,),
    in_specs=[_row_spec, _mat_spec, _vec_spec, _vec_spec],
    **_prep_outs,
)

_prepmid = pl.pallas_call(
    _prepmid_body,
    grid=(NRB,),
    in_specs=[_acc_spec, _den_spec, _vec_spec, _mat_spec, _vec_spec, _vec_spec],
    **_prep_outs,
)

_final = pl.pallas_call(
    _final_body,
    grid=(NRB,),
    in_specs=[_acc_spec, _den_spec, _vec_spec, _mat_spec, _vec_spec,
              _mat_spec, _vec_spec],
    out_specs=_row_spec,
    out_shape=jax.ShapeDtypeStruct((N, D), _f32),
)


_sc_mesh = plsc.VectorSubcoreMesh(core_axis_name="c", subcore_axis_name="s")

_sc_cp = pltpu.CompilerParams()
if "needs_layout_passes" in pltpu.CompilerParams.__dataclass_fields__:
    _sc_cp = dataclasses.replace(_sc_cp, needs_layout_passes=False)


@functools.partial(
    pl.kernel,
    compiler_params=_sc_cp,
    out_type=[
        jax.ShapeDtypeStruct((NC, N, D), _f32),
        jax.ShapeDtypeStruct((NC * N,), _f32),
    ],
    mesh=_sc_mesh,
    scratch_types=[
        pltpu.VMEM((2, B), _f32),
        pltpu.VMEM((2, B), _f32),
        pltpu.VMEM((2, B), jnp.int32),
        pltpu.VMEM((2, B), jnp.int32),
        pltpu.VMEM((2, B), _f32),
        pltpu.VMEM((2, B, D), _f32),
        pltpu.VMEM((640,), _f32),
        pltpu.VMEM((1, D), _f32),
        pltpu.VMEM_SHARED((NPAD, D), _f32),
        pltpu.VMEM_SHARED((NPAD,), _f32),
        pltpu.SemaphoreType.DMA,
        pltpu.SemaphoreType.DMA,
        pltpu.SemaphoreType.DMA,
        pltpu.SemaphoreType.DMA,
    ],
)
def _edge_pass(h_hbm, asrc_hbm, adst_hbm, srcp_hbm, dstp_hbm, m_hbm, zacc_hbm,
               accs_out, dens_out,
               asrc_v, adst_v, src_v, dst_v, ex_v, rows_v, den_stage, m_s,
               acc_sh, den_sh, gsem0, gsem1, asem0, asem1):
    cid = lax.axis_index("c")
    sid = lax.axis_index("s")
    wid = sid * NC + cid

    pltpu.sync_copy(m_hbm, m_s)
    m = m_s[0, pl.ds(0, L)][0]

    zr = NPAD // NS
    pltpu.sync_copy(zacc_hbm.at[pl.ds(sid * zr, zr)],
                    acc_sh.at[pl.ds(sid * zr, zr)])
    @pl.loop(0, 40)
    def _(j):
        den_stage[pl.ds(j * L, L)] = jnp.zeros((L,), _f32)

    pltpu.sync_copy(den_stage.at[pl.ds(0, zr)], den_sh.at[pl.ds(sid * zr, zr)])

    plsc.subcore_barrier()

    gsem = (gsem0, gsem1)
    asem = (asem0, asem1)
    ebase = wid * (NBLK * B)

    def _start_loads(b, q):
        eb = ebase + b * B
        pltpu.sync_copy(srcp_hbm.at[pl.ds(eb, B)], src_v.at[q])
        pltpu.sync_copy(dstp_hbm.at[pl.ds(eb, B)], dst_v.at[q])
        pltpu.async_copy(h_hbm.at[src_v.at[q]], rows_v.at[q], gsem[q])
        pltpu.async_copy(asrc_hbm.at[src_v.at[q]], asrc_v.at[q], asem[q])
        pltpu.async_copy(adst_hbm.at[dst_v.at[q]], adst_v.at[q], asem[q])

    def _process(b, p):
        q = 1 - p

        @pl.when(b + 1 < NBLK)
        def _():
            _start_loads(b + 1, q)

        pltpu.make_async_copy(asrc_hbm.at[src_v.at[p]], asrc_v.at[p],
                              asem[p]).wait()
        pltpu.make_async_copy(adst_hbm.at[dst_v.at[p]], adst_v.at[p],
                              asem[p]).wait()

        @pl.loop(0, B // L)
        def _(g):
            sl = pl.ds(g * L, L)
            e = asrc_v[p, sl] + adst_v[p, sl]
            e = jnp.where(e >= 0.0, e, 0.2 * e)
            ex_v[p, sl] = jnp.exp(e - m)

        pltpu.make_async_copy(h_hbm.at[src_v.at[p]], rows_v.at[p],
                              gsem[p]).wait()

        @pl.loop(0, B // L)
        def _(g):
            exg = ex_v[p, pl.ds(g * L, L)]
            for k in range(L):
                j = g * L + k
                exv = jnp.full((L,), exg[k], _f32)
                for c in range(D // L):
                    sl = pl.ds(c * L, L)
                    rows_v[p, j, sl] = rows_v[p, j, sl] * exv

        pltpu.sync_copy(rows_v.at[p], acc_sh.at[dst_v.at[p]], add=True)
        pltpu.sync_copy(ex_v.at[p], den_sh.at[dst_v.at[p]], add=True)

    _start_loads(0, 0)

    @pl.loop(0, NBLK, step=2)
    def _(blk):
        _process(blk, 0)
        _process(blk + 1, 1)

    plsc.subcore_barrier()

    orows = (N // NS) // 8 * 8
    pltpu.sync_copy(acc_sh.at[pl.ds(sid * orows, orows)],
                    accs_out.at[cid, pl.ds(sid * orows, orows)])

    @pl.when(sid == 0)
    def _():
        pltpu.sync_copy(acc_sh.at[pl.ds(NS * orows, N - NS * orows)],
                        accs_out.at[cid, pl.ds(NS * orows, N - NS * orows)])

    pltpu.sync_copy(den_sh.at[pl.ds(sid * orows, orows)],
                    den_stage.at[pl.ds(0, orows)])
    pltpu.sync_copy(den_stage.at[pl.ds(0, orows)],
                    dens_out.at[pl.ds(cid * N + sid * orows, orows)])

    @pl.when(sid == 0)
    def _():
        rem = N - NS * orows
        pltpu.sync_copy(den_sh.at[pl.ds(NS * orows, rem)],
                        den_stage.at[pl.ds(orows, rem)])
        pltpu.sync_copy(den_stage.at[pl.ds(orows, rem)],
                        dens_out.at[pl.ds(cid * N + NS * orows, rem)])


def _edge(h, asrc, adst, srcp, dstp, mv, zacc):
    asrc_e = jnp.pad(asrc.reshape(N), (0, TRASH))
    adst_e = jnp.pad(adst.reshape(N), (0, TRASH))
    accs, dens_flat = _edge_pass(h, asrc_e, adst_e, srcp, dstp, mv, zacc)
    dens = dens_flat.reshape(NC, NRB, 1, RB)
    return accs, dens


def kernel(x, edge_index, W1, att_src1, att_dst1, b1, W2, att_src2, att_dst2,
           b2, W3, att_src3, att_dst3, b3, lin1_W, lin1_b, lin2_W, lin2_b):
    src = edge_index[0].astype(jnp.int32).reshape(NW, EPW)
    dst = edge_index[1].astype(jnp.int32).reshape(NW, EPW)
    pads = jnp.arange(PAD_PER_W, dtype=jnp.int32) % L
    padd = N + jnp.arange(PAD_PER_W, dtype=jnp.int32) % TRASH
    srcp = jnp.concatenate(
        [src, jnp.broadcast_to(pads[None], (NW, PAD_PER_W))],
        axis=1).reshape(NW * NBLK * B)
    dstp = jnp.concatenate(
        [dst, jnp.broadcast_to(padd[None], (NW, PAD_PER_W))],
        axis=1).reshape(NW * NBLK * B)
    zacc = jnp.zeros((NPAD, D), _f32)

    h, asrc, adst, mv = _prep1(x, W1, att_src1.reshape(1, D),
                               att_dst1.reshape(1, D))
    accs, dens = _edge(h, asrc, adst, srcp, dstp, mv, zacc)
    h, asrc, adst, mv = _prepmid(accs, dens, b1.reshape(1, D), W2,
                                 att_src2.reshape(1, D), att_dst2.reshape(1, D))
    accs, dens = _edge(h, asrc, adst, srcp, dstp, mv, zacc)
    h, asrc, adst, mv = _prepmid(accs, dens, b2.reshape(1, D), W3,
                                 att_src3.reshape(1, D), att_dst3.reshape(1, D))
    accs, dens = _edge(h, asrc, adst, srcp, dstp, mv, zacc)
    out = _final(accs, dens, b3.reshape(1, D), lin1_W, lin1_b.reshape(1, D),
                 lin2_W, lin2_b.reshape(1, D))
    return out


# speedup vs baseline: 38.1174x; 1.0185x over previous
import dataclasses
import functools

import jax
import jax.numpy as jnp
from jax import lax
from jax.experimental import pallas as pl
from jax.experimental.pallas import tpu as pltpu
from jax.experimental.pallas import tpu_sc as plsc

N = 10000
E = 320000
D = 128
NC = 2
NS = 16
L = 16
NW = NC * NS
EPW = E // NW
B = 128
NBLK = 80
PAD_PER_W = NBLK * B - EPW
NPAD = 10112
TRASH = NPAD - N
RB = 1000
NRB = N // RB
SHIFT = 30.0

_f32 = jnp.float32


def _attention_stats(h, asv, adv, i, asrc_ref, adst_ref, m_ref, sa_ref, sd_ref):
    a_s = jnp.sum(h * asv, axis=1)
    a_d = jnp.sum(h * adv, axis=1)
    asrc_ref[...] = a_s.reshape(1, 1, RB)
    adst_ref[...] = a_d.reshape(1, 1, RB)
    bs = jnp.max(a_s)
    bd = jnp.max(a_d)

    @pl.when(i == 0)
    def _():
        sa_ref[0] = bs
        sd_ref[0] = bd

    @pl.when(i > 0)
    def _():
        sa_ref[0] = jnp.maximum(sa_ref[0], bs)
        sd_ref[0] = jnp.maximum(sd_ref[0], bd)

    @pl.when(i == NRB - 1)
    def _():
        m_ref[...] = jnp.full((1, D), sa_ref[0] + sd_ref[0] - SHIFT, _f32)


def _prep1_body(x_ref, w_ref, asv_ref, adv_ref,
                h_ref, asrc_ref, adst_ref, m_ref, sa_ref, sd_ref):
    i = pl.program_id(0)
    h = jnp.dot(x_ref[...], w_ref[...], preferred_element_type=_f32)
    h_ref[...] = h
    _attention_stats(h, asv_ref[...], adv_ref[...], i,
                     asrc_ref, adst_ref, m_ref, sa_ref, sd_ref)


def _prepmid_body(acc_ref, den_ref, b_ref, w_ref, asv_ref, adv_ref,
                  h_ref, asrc_ref, adst_ref, m_ref, sa_ref, sd_ref):
    i = pl.program_id(0)
    a = acc_ref[0] + acc_ref[1]
    dnm = den_ref[0, 0, 0] + den_ref[1, 0, 0] + 1e-16
    z = jnp.maximum(a / dnm[:, None] + b_ref[...], 0.0)
    h = jnp.dot(z, w_ref[...], preferred_element_type=_f32)
    h_ref[...] = h
    _attention_stats(h, asv_ref[...], adv_ref[...], i,
                     asrc_ref, adst_ref, m_ref, sa_ref, sd_ref)


def _final_body(acc_ref, den_ref, b_ref, w1_ref, b1_ref, w2_ref, b2_ref, out_ref):
    a = acc_ref[0] + acc_ref[1]
    dnm = den_ref[0, 0, 0] + den_ref[1, 0, 0] + 1e-16
    z = jnp.maximum(a / dnm[:, None] + b_ref[...], 0.0)
    t = jnp.maximum(
        jnp.dot(z, w1_ref[...], preferred_element_type=_f32) + b1_ref[...], 0.0)
    out_ref[...] = (
        jnp.dot(t, w2_ref[...], preferred_element_type=_f32) + b2_ref[...])


_row_spec = pl.BlockSpec((RB, D), lambda i: (i, 0))
_mat_spec = pl.BlockSpec((D, D), lambda i: (0, 0))
_vec_spec = pl.BlockSpec((1, D), lambda i: (0, 0))
_a_spec = pl.BlockSpec((1, 1, RB), lambda i: (i, 0, 0))
_acc_spec = pl.BlockSpec((NC, RB, D), lambda i: (0, i, 0))
_den_spec = pl.BlockSpec((NC, 1, 1, RB), lambda i: (0, i, 0, 0))

_prep_outs = dict(
    out_specs=[_row_spec, _a_spec, _a_spec, _vec_spec],
    out_shape=[
        jax.ShapeDtypeStruct((N, D), _f32),
        jax.ShapeDtypeStruct((NRB, 1, RB), _f32),
        jax.ShapeDtypeStruct((NRB, 1, RB), _f32),
        jax.ShapeDtypeStruct((1, D), _f32),
    ],
    scratch_shapes=[pltpu.SMEM((1,), _f32), pltpu.SMEM((1,), _f32)],
)

_prep1 = pl.pallas_call(
    _prep1_body,
    grid=(NRB,),
    in_specs=[_row_spec, _mat_spec, _vec_spec, _vec_spec],
    **_prep_outs,
)

_prepmid = pl.pallas_call(
    _prepmid_body,
    grid=(NRB,),
    in_specs=[_acc_spec, _den_spec, _vec_spec, _mat_spec, _vec_spec, _vec_spec],
    **_prep_outs,
)

_final = pl.pallas_call(
    _final_body,
    grid=(NRB,),
    in_specs=[_acc_spec, _den_spec, _vec_spec, _mat_spec, _vec_spec,
              _mat_spec, _vec_spec],
    out_specs=_row_spec,
    out_shape=jax.ShapeDtypeStruct((N, D), _f32),
)


_sc_mesh = plsc.VectorSubcoreMesh(core_axis_name="c", subcore_axis_name="s")

_sc_cp = pltpu.CompilerParams()
if "needs_layout_passes" in pltpu.CompilerParams.__dataclass_fields__:
    _sc_cp = dataclasses.replace(_sc_cp, needs_layout_passes=False)


@functools.partial(
    pl.kernel,
    compiler_params=_sc_cp,
    out_type=[
        jax.ShapeDtypeStruct((NC, N, D), _f32),
        jax.ShapeDtypeStruct((NC * N,), _f32),
    ],
    mesh=_sc_mesh,
    scratch_types=[
        pltpu.VMEM((2, B), _f32),
        pltpu.VMEM((2, B), _f32),
        pltpu.VMEM((2, B), jnp.int32),
        pltpu.VMEM((2, B), jnp.int32),
        pltpu.VMEM((2, B), _f32),
        pltpu.VMEM((2, B, D), _f32),
        pltpu.VMEM((640,), _f32),
        pltpu.VMEM((1, D), _f32),
        pltpu.VMEM_SHARED((NPAD, D), _f32),
        pltpu.VMEM_SHARED((NPAD,), _f32),
        pltpu.SemaphoreType.DMA,
        pltpu.SemaphoreType.DMA,
        pltpu.SemaphoreType.DMA,
        pltpu.SemaphoreType.DMA,
        pltpu.SemaphoreType.DMA,
        pltpu.SemaphoreType.DMA,
    ],
)
def _edge_pass(h_hbm, asrc_hbm, adst_hbm, srcp_hbm, dstp_hbm, m_hbm, zacc_hbm,
               accs_out, dens_out,
               asrc_v, adst_v, src_v, dst_v, ex_v, rows_v, den_stage, m_s,
               acc_sh, den_sh, gsem0, gsem1, asem0, asem1, ssem0, ssem1):
    cid = lax.axis_index("c")
    sid = lax.axis_index("s")
    wid = sid * NC + cid

    pltpu.sync_copy(m_hbm, m_s)
    m = m_s[0, pl.ds(0, L)][0]

    zr = NPAD // NS
    pltpu.sync_copy(zacc_hbm.at[pl.ds(sid * zr, zr)],
                    acc_sh.at[pl.ds(sid * zr, zr)])
    @pl.loop(0, 40)
    def _(j):
        den_stage[pl.ds(j * L, L)] = jnp.zeros((L,), _f32)

    pltpu.sync_copy(den_stage.at[pl.ds(0, zr)], den_sh.at[pl.ds(sid * zr, zr)])

    plsc.subcore_barrier()

    gsem = (gsem0, gsem1)
    asem = (asem0, asem1)
    ssem = (ssem0, ssem1)
    ebase = wid * (NBLK * B)

    def _start_loads(b, q):
        eb = ebase + b * B
        pltpu.sync_copy(srcp_hbm.at[pl.ds(eb, B)], src_v.at[q])
        pltpu.sync_copy(dstp_hbm.at[pl.ds(eb, B)], dst_v.at[q])
        pltpu.async_copy(h_hbm.at[src_v.at[q]], rows_v.at[q], gsem[q])
        pltpu.async_copy(asrc_hbm.at[src_v.at[q]], asrc_v.at[q], asem[q])
        pltpu.async_copy(adst_hbm.at[dst_v.at[q]], adst_v.at[q], asem[q])

    def _process(b, p):
        q = 1 - p

        @pl.when(b + 1 < NBLK)
        def _():
            @pl.when(b >= 1)
            def _():
                pltpu.make_async_copy(rows_v.at[q], acc_sh.at[dst_v.at[q]],
                                      ssem[q]).wait()
                pltpu.make_async_copy(ex_v.at[q], den_sh.at[dst_v.at[q]],
                                      ssem[q]).wait()

            _start_loads(b + 1, q)

        pltpu.make_async_copy(asrc_hbm.at[src_v.at[p]], asrc_v.at[p],
                              asem[p]).wait()
        pltpu.make_async_copy(adst_hbm.at[dst_v.at[p]], adst_v.at[p],
                              asem[p]).wait()

        @pl.loop(0, B // L)
        def _(g):
            sl = pl.ds(g * L, L)
            e = asrc_v[p, sl] + adst_v[p, sl]
            e = jnp.where(e >= 0.0, e, 0.2 * e)
            ex_v[p, sl] = jnp.exp(e - m)

        pltpu.make_async_copy(h_hbm.at[src_v.at[p]], rows_v.at[p],
                              gsem[p]).wait()

        @pl.loop(0, B // L)
        def _(g):
            exg = ex_v[p, pl.ds(g * L, L)]
            for k in range(L):
                j = g * L + k
                exv = jnp.full((L,), exg[k], _f32)
                for c in range(D // L):
                    sl = pl.ds(c * L, L)
                    rows_v[p, j, sl] = rows_v[p, j, sl] * exv

        pltpu.async_copy(rows_v.at[p], acc_sh.at[dst_v.at[p]], ssem[p],
                         add=True)
        pltpu.async_copy(ex_v.at[p], den_sh.at[dst_v.at[p]], ssem[p],
                         add=True)

    _start_loads(0, 0)

    @pl.loop(0, NBLK, step=2)
    def _(blk):
        _process(blk, 0)
        _process(blk + 1, 1)

    for p in (0, 1):
        pltpu.make_async_copy(rows_v.at[p], acc_sh.at[dst_v.at[p]],
                              ssem[p]).wait()
        pltpu.make_async_copy(ex_v.at[p], den_sh.at[dst_v.at[p]],
                              ssem[p]).wait()

    plsc.subcore_barrier()

    orows = (N // NS) // 8 * 8
    pltpu.sync_copy(acc_sh.at[pl.ds(sid * orows, orows)],
                    accs_out.at[cid, pl.ds(sid * orows, orows)])

    @pl.when(sid == 0)
    def _():
        pltpu.sync_copy(acc_sh.at[pl.ds(NS * orows, N - NS * orows)],
                        accs_out.at[cid, pl.ds(NS * orows, N - NS * orows)])

    pltpu.sync_copy(den_sh.at[pl.ds(sid * orows, orows)],
                    den_stage.at[pl.ds(0, orows)])
    pltpu.sync_copy(den_stage.at[pl.ds(0, orows)],
                    dens_out.at[pl.ds(cid * N + sid * orows, orows)])

    @pl.when(sid == 0)
    def _():
        rem = N - NS * orows
        pltpu.sync_copy(den_sh.at[pl.ds(NS * orows, rem)],
                        den_stage.at[pl.ds(orows, rem)])
        pltpu.sync_copy(den_stage.at[pl.ds(orows, rem)],
                        dens_out.at[pl.ds(cid * N + NS * orows, rem)])


def _edge(h, asrc, adst, srcp, dstp, mv, zacc):
    asrc_e = jnp.pad(asrc.reshape(N), (0, TRASH))
    adst_e = jnp.pad(adst.reshape(N), (0, TRASH))
    accs, dens_flat = _edge_pass(h, asrc_e, adst_e, srcp, dstp, mv, zacc)
    dens = dens_flat.reshape(NC, NRB, 1, RB)
    return accs, dens


def kernel(x, edge_index, W1, att_src1, att_dst1, b1, W2, att_src2, att_dst2,
           b2, W3, att_src3, att_dst3, b3, lin1_W, lin1_b, lin2_W, lin2_b):
    src = edge_index[0].astype(jnp.int32).reshape(NW, EPW)
    dst = edge_index[1].astype(jnp.int32).reshape(NW, EPW)
    pads = jnp.arange(PAD_PER_W, dtype=jnp.int32) % L
    padd = N + jnp.arange(PAD_PER_W, dtype=jnp.int32) % TRASH
    srcp = jnp.concatenate(
        [src, jnp.broadcast_to(pads[None], (NW, PAD_PER_W))],
        axis=1).reshape(NW * NBLK * B)
    dstp = jnp.concatenate(
        [dst, jnp.broadcast_to(padd[None], (NW, PAD_PER_W))],
        axis=1).reshape(NW * NBLK * B)
    zacc = jnp.zeros((NPAD, D), _f32)

    h, asrc, adst, mv = _prep1(x, W1, att_src1.reshape(1, D),
                               att_dst1.reshape(1, D))
    accs, dens = _edge(h, asrc, adst, srcp, dstp, mv, zacc)
    h, asrc, adst, mv = _prepmid(accs, dens, b1.reshape(1, D), W2,
                                 att_src2.reshape(1, D), att_dst2.reshape(1, D))
    accs, dens = _edge(h, asrc, adst, srcp, dstp, mv, zacc)
    h, asrc, adst, mv = _prepmid(accs, dens, b2.reshape(1, D), W3,
                                 att_src3.reshape(1, D), att_dst3.reshape(1, D))
    accs, dens = _edge(h, asrc, adst, srcp, dstp, mv, zacc)
    out = _final(accs, dens, b3.reshape(1, D), lin1_W, lin1_b.reshape(1, D),
                 lin2_W, lin2_b.reshape(1, D))
    return out
